# Initial kernel scaffold; baseline (speedup 1.0000x reference)
import jax
import jax.numpy as jnp
from jax.experimental import pallas as pl


def kernel(x, ew0, eb0, ew1, eb1, ew2, eb2, ew3, eb3, dw0, db0, dw1, db1, dw2, db2, dw3, db3, emb, ema_cs):
    raise NotImplementedError("write your pallas kernel here")


# speedup vs baseline: 1.0944x; 1.0944x over previous
import jax
import jax.numpy as jnp
from jax import lax
from jax.experimental import pallas as pl

F32 = jnp.float32
BF16 = jnp.bfloat16
_INV_SQRT2 = 0.7071067811865476


def _gelu(v):
    return v * 0.5 * (1.0 + lax.erf(v * _INV_SQRT2))


def _nhwc(t):
    return jnp.transpose(t, (0, 2, 3, 1))


def _prep_w(w):
    return jnp.transpose(w, (2, 3, 1, 0))


def _pad_sp(t, p):
    return jnp.pad(t, ((0, 0), (p, p), (p, p), (0, 0)))


def _conv_s1(xpad, w, b, act):
    B, Hp, Wp, C = xpad.shape
    HO, WO = Hp - 2, Wp - 2
    Co = w.shape[-1]
    M = HO * WO

    nch = 4 if (HO % 4 == 0 and HO >= 32) else 1
    RH = HO // nch

    def body(x_ref, w_ref, b_ref, o_ref):
        for rr in range(nch):
            r0 = rr * RH
            acc = jnp.zeros((RH * WO, Co), F32)
            for dy in range(3):
                for dx in range(3):
                    xs = x_ref[0, r0 + dy:r0 + dy + RH, dx:dx + WO, :]
                    acc = acc + jnp.dot(
                        xs.reshape(RH * WO, C).astype(BF16),
                        w_ref[dy, dx].astype(BF16),
                        preferred_element_type=F32)
            r = acc + b_ref[...]
            if act:
                r = _gelu(r)
            o_ref[0, r0:r0 + RH, :, :] = r.reshape(RH, WO, Co)

    return pl.pallas_call(
        body,
        grid=(B,),
        in_specs=[pl.BlockSpec((1, Hp, Wp, C), lambda i: (i, 0, 0, 0)),
                  pl.BlockSpec((3, 3, C, Co), lambda i: (0, 0, 0, 0)),
                  pl.BlockSpec((1, Co), lambda i: (0, 0))],
        out_specs=pl.BlockSpec((1, HO, WO, Co), lambda i: (i, 0, 0, 0)),
        out_shape=jax.ShapeDtypeStruct((B, HO, WO, Co), F32),
    )(xpad, w, b.reshape(1, Co))


def _vq(zf, emb, e2_row, cs_row):
    M, C = zf.shape
    V = emb.shape[0]
    G = 4 if M % 4 == 0 else 1
    BM = M // G

    def body(z_ref, e_ref, e2_ref, cs_ref, q_ref, p_ref, l_ref):
        z = z_ref[...]
        e = e_ref[...]
        zb = lax.dot_general(z.astype(BF16), e.astype(BF16),
                             (((1,), (1,)), ((), ())),
                             preferred_element_type=F32)
        z2 = jnp.sum(z * z, axis=1, keepdims=True)
        d = z2 - 2.0 * zb + e2_ref[...]
        idx = jnp.argmin(d, axis=1)
        iota = lax.broadcasted_iota(jnp.int32, (BM, V), 1)
        onehot = (iota == idx[:, None]).astype(F32)
        q_ref[...] = jnp.dot(onehot, e, preferred_element_type=F32,
                             precision=jax.lax.Precision.HIGHEST)
        cs = cs_ref[...]
        probs = cs / jnp.sum(cs)
        p_ref[...] = jnp.sum(onehot * probs, axis=1, keepdims=True)
        dmin = jnp.min(d, axis=1)

        @pl.when(pl.program_id(0) == 0)
        def _():
            l_ref[...] = jnp.zeros((1, 1), F32)

        l_ref[...] += jnp.sum(dmin)[None, None]

    return pl.pallas_call(
        body,
        grid=(G,),
        in_specs=[pl.BlockSpec((BM, C), lambda i: (i, 0)),
                  pl.BlockSpec((V, C), lambda i: (0, 0)),
                  pl.BlockSpec((1, V), lambda i: (0, 0)),
                  pl.BlockSpec((1, V), lambda i: (0, 0))],
        out_specs=[pl.BlockSpec((BM, C), lambda i: (i, 0)),
                   pl.BlockSpec((BM, 1), lambda i: (i, 0)),
                   pl.BlockSpec((1, 1), lambda i: (0, 0))],
        out_shape=[jax.ShapeDtypeStruct((M, C), F32),
                   jax.ShapeDtypeStruct((M, 1), F32),
                   jax.ShapeDtypeStruct((1, 1), F32)],
    )(zf, emb, e2_row, cs_row)


def _dec0(xpad, w, b):
    B, Hp, Wp, C = xpad.shape
    HI, WI = Hp - 2, Wp - 2
    Co = w.shape[-1]
    M = HI * WI

    def off(k):
        return (2 - k) // 2 + 1 if k % 2 == 0 else (3 - k) // 2 + 1

    def body(x_ref, w_ref, b_ref, o_ref):
        for py in range(2):
            khs = (0, 2, 4) if py == 0 else (1, 3)
            for px in range(2):
                kws = (0, 2, 4) if px == 0 else (1, 3)
                acc = jnp.zeros((M, Co), F32)
                for kh in khs:
                    oy = off(kh)
                    for kw in kws:
                        ox = off(kw)
                        xs = x_ref[0, oy:oy + HI, ox:ox + WI, :]
                        acc = acc + jnp.dot(
                            xs.reshape(M, C).astype(BF16),
                            w_ref[kh, kw].astype(BF16),
                            preferred_element_type=F32)
                o_ref[0, py, px] = _gelu(acc + b_ref[...]).reshape(HI, WI, Co)

    return pl.pallas_call(
        body,
        grid=(B,),
        in_specs=[pl.BlockSpec((1, Hp, Wp, C), lambda i: (i, 0, 0, 0)),
                  pl.BlockSpec((5, 5, C, Co), lambda i: (0, 0, 0, 0)),
                  pl.BlockSpec((1, Co), lambda i: (0, 0))],
        out_specs=pl.BlockSpec((1, 2, 2, HI, WI, Co),
                               lambda i: (i, 0, 0, 0, 0, 0)),
        out_shape=jax.ShapeDtypeStruct((B, 2, 2, HI, WI, Co), F32),
    )(xpad, w, b.reshape(1, Co))


def _dec23(xpad, w2, b2, w3, b3):
    B, Hp, Wp, C = xpad.shape
    HO, WO = Hp - 2, Wp - 2
    Cm = w2.shape[-1]
    Co = w3.shape[-1]
    M = HO * WO

    nch = 4 if (HO % 4 == 0 and HO >= 32) else 1
    RH = HO // nch

    def body(x_ref, w2_ref, b2_ref, w3_ref, b3_ref, o_ref):
        o_ref[...] = jnp.broadcast_to(b3_ref[...].reshape(1, 1, 1, Co),
                                      (1, Hp, Wp, Co))
        for rr in range(nch):
            r0 = rr * RH
            acc = jnp.zeros((RH * WO, Cm), F32)
            for dy in range(3):
                for dx in range(3):
                    xs = x_ref[0, r0 + dy:r0 + dy + RH, dx:dx + WO, :]
                    acc = acc + jnp.dot(
                        xs.reshape(RH * WO, C).astype(BF16),
                        w2_ref[dy, dx].astype(BF16),
                        preferred_element_type=F32)
            g = _gelu(acc + b2_ref[...])
            y = jnp.dot(g.astype(BF16), w3_ref[...].astype(BF16),
                        preferred_element_type=F32) + b3_ref[...]
            o_ref[0, 1 + r0:1 + r0 + RH, 1:1 + WO, :] = y.reshape(RH, WO, Co)

    return pl.pallas_call(
        body,
        grid=(B,),
        in_specs=[pl.BlockSpec((1, Hp, Wp, C), lambda i: (i, 0, 0, 0)),
                  pl.BlockSpec((3, 3, C, Cm), lambda i: (0, 0, 0, 0)),
                  pl.BlockSpec((1, Cm), lambda i: (0, 0)),
                  pl.BlockSpec((Cm, Co), lambda i: (0, 0)),
                  pl.BlockSpec((1, Co), lambda i: (0, 0))],
        out_specs=pl.BlockSpec((1, Hp, Wp, Co), lambda i: (i, 0, 0, 0)),
        out_shape=jax.ShapeDtypeStruct((B, Hp, Wp, Co), F32),
    )(xpad, w2, b2.reshape(1, Cm), w3, b3.reshape(1, Co))


def _enc_conv(x, w, b, stride, pad):
    out = lax.conv_general_dilated(
        x, w, (stride, stride), [(pad, pad), (pad, pad)],
        dimension_numbers=('NCHW', 'OIHW', 'NCHW'))
    return out + b[None, :, None, None]


def kernel(x, ew0, eb0, ew1, eb1, ew2, eb2, ew3, eb3,
           dw0, db0, dw1, db1, dw2, db2, dw3, db3, emb, ema_cs):
    x = x.astype(F32)

    g = lambda v: jax.nn.gelu(v, approximate=False)
    h = g(_enc_conv(x, ew0, eb0, 2, 2))
    h = g(_enc_conv(h, ew1, eb1, 2, 2))
    h = g(_enc_conv(h, ew2, eb2, 2, 2))
    z = _enc_conv(h, ew3, eb3, 1, 1)

    Bz, Cz, Hz, Wz = z.shape
    M = Bz * Hz * Wz
    zf = jnp.transpose(z, (0, 2, 3, 1)).reshape(-1, Cz)
    V = emb.shape[0]
    e2_row = jnp.sum(emb * emb, axis=1)[None, :]
    q, zp, lsum = _vq(zf, emb, e2_row, ema_cs.reshape(1, V))
    vq_loss = (0.25 / (M * Cz)) * lsum[0, 0]
    z_probs = zp.reshape(Bz, Hz, Wz)
    qim = q.reshape(Bz, Hz, Wz, Cz)

    hp = _dec0(_pad_sp(qim, 1), jnp.transpose(dw0, (2, 3, 0, 1)), db0)
    T = hp.shape[-1]
    h = jnp.transpose(hp, (0, 3, 1, 4, 2, 5)).reshape(Bz, 2 * Hz, 2 * Wz, T)
    h = _conv_s1(_pad_sp(h, 1), _prep_w(dw1), db1, act=True)
    w3 = jnp.transpose(dw3[:, :, 0, 0], (1, 0))
    xh = _dec23(_pad_sp(h, 1), _prep_w(dw2), db2, w3, db3)
    x_hat = jnp.transpose(xh, (0, 3, 1, 2))
    return (x_hat, z_probs, vq_loss)
